# baseline (speedup 1.0000x reference)
import jax
import jax.numpy as jnp
from jax import lax
from jax.experimental import pallas as pl
from jax.experimental.pallas import tpu as pltpu
from jax.experimental.pallas import tpu_sc as plsc

B = 16384
D = 64
NC = 2
NS = 16
L = 16
NW = NC * NS
BPW = B // NW
CHUNK = 128
NCHUNK = BPW // CHUNK
GROUPS = BPW // L
DV = D // L


def _sc_body(ent_hbm, rel_hbm, hidx_hbm, ridx_hbm, tidx_hbm, out_hbm,
             hidx_v, ridx_v, tidx_v, hrows, rrows, trows, scores, sem):
    wid = lax.axis_index("s") * NC + lax.axis_index("c")
    base = wid * BPW

    pltpu.sync_copy(hidx_hbm.at[pl.ds(base, BPW)], hidx_v)
    pltpu.sync_copy(ridx_hbm.at[pl.ds(base, BPW)], ridx_v)
    pltpu.sync_copy(tidx_hbm.at[pl.ds(base, BPW)], tidx_v)

    copies = []
    for j in range(NCHUNK):
        sl = pl.ds(j * CHUNK, CHUNK)
        copies.append(pltpu.make_async_copy(
            ent_hbm.at[hidx_v.at[sl]], hrows.at[sl], sem))
        copies.append(pltpu.make_async_copy(
            rel_hbm.at[ridx_v.at[sl]], rrows.at[sl], sem))
        copies.append(pltpu.make_async_copy(
            ent_hbm.at[tidx_v.at[sl]], trows.at[sl], sem))
    for c in copies:
        c.start()
    for c in copies:
        c.wait()

    lane = lax.iota(jnp.int32, L)

    def group(g, carry):
        res = jnp.zeros((L,), jnp.float32)
        for j in range(L):
            i = g * L + j
            s = jnp.zeros((L,), jnp.float32)
            for c in range(DV):
                sl = pl.ds(c * L, L)
                s = s + hrows[i, sl] * rrows[i, sl] * trows[i, sl]
            total = jnp.sum(s)
            res = jnp.where(lane == j, total, res)
        scores[pl.ds(g * L, L)] = res
        return carry

    lax.fori_loop(0, GROUPS, group, 0)

    pltpu.sync_copy(scores, out_hbm.at[pl.ds(base, BPW)])


@jax.jit
def _score(hidx, ridx, tidx, entity_embedding, relation_embedding):
    mesh = plsc.VectorSubcoreMesh(core_axis_name="c", subcore_axis_name="s",
                                  num_cores=NC, num_subcores=NS)
    kern = pl.kernel(
        _sc_body,
        out_type=jax.ShapeDtypeStruct((B,), jnp.float32),
        mesh=mesh,
        compiler_params=pltpu.CompilerParams(needs_layout_passes=False,
                                             use_tc_tiling_on_sc=False),
        scratch_types=[
            pltpu.VMEM((BPW,), jnp.int32),
            pltpu.VMEM((BPW,), jnp.int32),
            pltpu.VMEM((BPW,), jnp.int32),
            pltpu.VMEM((BPW, D), jnp.float32),
            pltpu.VMEM((BPW, D), jnp.float32),
            pltpu.VMEM((BPW, D), jnp.float32),
            pltpu.VMEM((BPW,), jnp.float32),
            pltpu.SemaphoreType.DMA,
        ],
    )
    return kern(entity_embedding, relation_embedding, hidx, ridx, tidx)


def kernel(sample, entity_embedding, relation_embedding):
    sample = sample.astype(jnp.int32)
    hidx = sample[:, 0]
    ridx = sample[:, 1]
    tidx = sample[:, 2]
    out = _score(hidx, ridx, tidx, entity_embedding, relation_embedding)
    return out[:, None]
